# baseline (speedup 1.0000x reference)
import jax
import jax.numpy as jnp
from jax import lax
from jax.experimental import pallas as pl
from jax.experimental.pallas import tpu as pltpu
from jax.experimental.pallas import tpu_sc as plsc

VOCAB = 1000000
EMB = 32
B = 4096
L = 50
NPAIR = 5
BL = B * L
NUNIT = NPAIR * L
NROW = VOCAB * EMB // 128

VC = 800
NCH = VOCAB // VC
OR = VC * EMB // 128

BC = 128
NBC = B // BC

NC, NS = 2, 16
NW = NC * NS

_params = pltpu.CompilerParams(needs_layout_passes=False)


def _tr_kernel(w1c, w1b, outc, outb, in0, in1, o0, o1, isem0, isem1, wsem0, wsem1):
    wid = lax.axis_index("s") * NC + lax.axis_index("c")
    iotav = lax.iota(jnp.int32, 16) * VC

    def table(w1, out):
        nu = (NCH - wid + NW - 1) // NW

        def stage(i, inb, sem):
            c = wid + i * NW
            v0 = pl.multiple_of(c * VC, 8)
            for e in range(EMB):
                pltpu.async_copy(w1.at[pl.ds(v0 + e * VOCAB, VC)],
                                 inb.at[pl.ds(e * VC, VC)], sem)

        def drain_stage(sem):
            for e in range(EMB):
                pltpu.make_async_copy(w1.at[pl.ds(0, VC)],
                                      in0.at[pl.ds(0, VC)], sem).wait()

        def transpose(inb, ob):
            def tbody(dv, carry):
                va = plsc.load_gather(inb, [iotav + dv])
                vb = plsc.load_gather(inb, [iotav + (dv + 16 * VC)])
                r = dv >> 2
                c0 = (dv & 3) * EMB
                ob[r, pl.ds(c0, 16)] = va
                ob[r, pl.ds(c0 + 16, 16)] = vb
                return carry

            lax.fori_loop(0, VC, tbody, 0, unroll=False)

        def write(i, ob, sem):
            c = wid + i * NW
            pltpu.async_copy(
                ob, out.at[pl.ds(pl.multiple_of(c * OR, 8), OR), :], sem)

        def drain_write(sem):
            pltpu.make_async_copy(o0, out.at[pl.ds(0, OR), :], sem).wait()

        stage(0, in0, isem0)

        def body(k, carry):
            for half, inb, ob, isem, wsem, isem_n in (
                    (0, in0, o0, isem0, wsem0, isem1),
                    (1, in1, o1, isem1, wsem1, isem0)):
                i = 2 * k + half

                @pl.when(i < nu)
                def _():
                    drain_stage(isem)

                    @pl.when(i + 1 < nu)
                    def _():
                        stage(i + 1, in1 if half == 0 else in0, isem_n)

                    @pl.when(i >= 2)
                    def _():
                        drain_write(wsem)

                    transpose(inb, ob)
                    write(i, ob, wsem)
            return carry

        lax.fori_loop(0, (nu + 1) // 2, body, 0, unroll=False)
        drain_write(wsem0)
        drain_write(wsem1)

    table(w1c, outc)
    table(w1b, outb)


def _lk_kernel(idx_hbm, wc, wb, out_hbm,
               idx_a, idx_b, ric0, ric1, rib0, rib1, ac0, ac1, ab0, ab1,
               dc0, dc1, db0, db1, ob0, ob1, isem, gsem0, gsem1, wsem0,
               wsem1):
    wid = lax.axis_index("s") * NC + lax.axis_index("c")
    iota = lax.iota(jnp.int32, 16)

    def do_unit(u, carry):
        p = u // L
        l = u % L
        ia = pltpu.async_copy(
            idx_hbm.at[pl.ds(pl.multiple_of(2 * p * BL + l * B, 8), B)],
            idx_a, isem)
        ib = pltpu.async_copy(
            idx_hbm.at[pl.ds(pl.multiple_of((2 * p + 1) * BL + l * B, 8), B)],
            idx_b, isem)
        ia.wait()
        ib.wait()

        def prep(q, ric, rib, ac, ab):
            def pbody(j, carry):
                o = q * BC + j * 16
                vc = idx_a[pl.ds(o, 16)]
                vb = idx_b[pl.ds(o, 16)]
                ric[pl.ds(j * 16, 16)] = lax.shift_right_logical(vc, 2)
                rib[pl.ds(j * 16, 16)] = lax.shift_right_logical(vb, 2)
                ac[pl.ds(j * 16, 16)] = lax.shift_left(
                    lax.bitwise_and(vc, 3), 5)
                ab[pl.ds(j * 16, 16)] = lax.shift_left(
                    lax.bitwise_and(vb, 3), 5)
                return carry

            lax.fori_loop(0, BC // 16, pbody, 0, unroll=False)

        def fire(ric, rib, dc, db, sem):
            pltpu.async_copy(wc.at[ric], dc, sem)
            pltpu.async_copy(wb.at[rib], db, sem)

        def drain_gathers(sem):
            pltpu.make_async_copy(wc.at[pl.ds(0, BC), :], dc0, sem).wait()
            pltpu.make_async_copy(wc.at[pl.ds(0, BC), :], dc0, sem).wait()

        def extract(ac, ab, dc, db, ob):
            def ebody(j, carry):
                rv = iota + j * 16
                va = ac[pl.ds(j * 16, 16)]
                vb = ab[pl.ds(j * 16, 16)]
                for e in range(EMB):
                    ob[e, pl.ds(j * 16, 16)] = plsc.load_gather(
                        dc, [rv, va + e])
                for e in range(EMB):
                    ob[EMB + e, pl.ds(j * 16, 16)] = plsc.load_gather(
                        db, [rv, vb + e])
                return carry

            lax.fori_loop(0, BC // 16, ebody, 0, unroll=False)

        def write(q, ob, sem):
            pltpu.async_copy(
                ob,
                out_hbm.at[p, l, :, pl.ds(pl.multiple_of(q * BC, 128), BC)],
                sem)

        def drain_write(sem):
            pltpu.make_async_copy(
                ob0, out_hbm.at[0, 0, :, pl.ds(0, BC)], sem).wait()

        prep(0, ric0, rib0, ac0, ab0)
        fire(ric0, rib0, dc0, db0, gsem0)

        def chunk_pair(m, carry):
            q0 = 2 * m
            prep(q0 + 1, ric1, rib1, ac1, ab1)
            fire(ric1, rib1, dc1, db1, gsem1)
            drain_gathers(gsem0)

            @pl.when(m >= 1)
            def _():
                drain_write(wsem0)

            extract(ac0, ab0, dc0, db0, ob0)
            write(q0, ob0, wsem0)

            @pl.when(q0 + 2 < NBC)
            def _():
                prep(q0 + 2, ric0, rib0, ac0, ab0)
                fire(ric0, rib0, dc0, db0, gsem0)

            drain_gathers(gsem1)

            @pl.when(m >= 1)
            def _():
                drain_write(wsem1)

            extract(ac1, ab1, dc1, db1, ob1)
            write(q0 + 1, ob1, wsem1)
            return carry

        lax.fori_loop(0, NBC // 2, chunk_pair, 0, unroll=False)
        drain_write(wsem0)
        drain_write(wsem1)
        return carry

    nu = (NUNIT - wid + NW - 1) // NW

    def body(k, carry):
        return do_unit(wid + k * NW, carry)

    lax.fori_loop(0, nu, body, 0, unroll=False)


@jax.jit
def _run(idx_all, w1c, w1b):
    mesh = plsc.VectorSubcoreMesh(core_axis_name="c", subcore_axis_name="s")
    tr = pl.kernel(
        _tr_kernel,
        out_type=(jax.ShapeDtypeStruct((NROW, 128), jnp.float32),
                  jax.ShapeDtypeStruct((NROW, 128), jnp.float32)),
        mesh=mesh,
        scratch_types=[
            pltpu.VMEM((EMB * VC,), jnp.float32),
            pltpu.VMEM((EMB * VC,), jnp.float32),
            pltpu.VMEM((OR, 128), jnp.float32),
            pltpu.VMEM((OR, 128), jnp.float32),
            pltpu.SemaphoreType.DMA,
            pltpu.SemaphoreType.DMA,
            pltpu.SemaphoreType.DMA,
            pltpu.SemaphoreType.DMA,
        ],
        compiler_params=_params,
    )
    wc, wb = tr(w1c, w1b)
    lk = pl.kernel(
        _lk_kernel,
        out_type=jax.ShapeDtypeStruct((NPAIR, L, 2 * EMB, B), jnp.float32),
        mesh=mesh,
        scratch_types=[
            pltpu.VMEM((B,), jnp.int32),
            pltpu.VMEM((B,), jnp.int32),
            pltpu.VMEM((BC,), jnp.int32),
            pltpu.VMEM((BC,), jnp.int32),
            pltpu.VMEM((BC,), jnp.int32),
            pltpu.VMEM((BC,), jnp.int32),
            pltpu.VMEM((BC,), jnp.int32),
            pltpu.VMEM((BC,), jnp.int32),
            pltpu.VMEM((BC,), jnp.int32),
            pltpu.VMEM((BC,), jnp.int32),
            pltpu.VMEM((BC, 128), jnp.float32),
            pltpu.VMEM((BC, 128), jnp.float32),
            pltpu.VMEM((BC, 128), jnp.float32),
            pltpu.VMEM((BC, 128), jnp.float32),
            pltpu.VMEM((2 * EMB, BC), jnp.float32),
            pltpu.VMEM((2 * EMB, BC), jnp.float32),
            pltpu.SemaphoreType.DMA,
            pltpu.SemaphoreType.DMA,
            pltpu.SemaphoreType.DMA,
            pltpu.SemaphoreType.DMA,
            pltpu.SemaphoreType.DMA,
        ],
        compiler_params=_params,
    )
    return lk(idx_all, wc, wb)


def kernel(idx0, idx1, idx2, idx3, idx4, idx5, idx6, idx7, idx8, idx9,
           W_cate, W_brand):
    idx_all = jnp.concatenate(
        [i.T.reshape(BL) for i in
         (idx0, idx1, idx2, idx3, idx4, idx5, idx6, idx7, idx8, idx9)])
    w1c = W_cate.T.reshape(VOCAB * EMB)
    w1b = W_brand.T.reshape(VOCAB * EMB)
    out_k = _run(idx_all, w1c, w1b)
    return jnp.transpose(out_k, (0, 3, 1, 2))


# speedup vs baseline: 2.6848x; 2.6848x over previous
import jax
import jax.numpy as jnp
from jax import lax
from jax.experimental import pallas as pl
from jax.experimental.pallas import tpu as pltpu
from jax.experimental.pallas import tpu_sc as plsc

VOCAB = 1000000
EMB = 32
B = 4096
L = 50
NPAIR = 5
NUNIT = NPAIR * L
NROW = VOCAB * EMB // 128

VC = 512
NCH = VOCAB // VC
VTAIL = VOCAB - (VOCAB // VC) * VC
TROWS = VTAIL * EMB // 128
OR = VC * EMB // 128

BC = 128
NBC = B // BC

NC, NS = 2, 16
NW = NC * NS

_params = pltpu.CompilerParams(needs_layout_passes=False)


def _tr_kernel(wct, wbt, tailc, tailb, outc, outb, s0, s1, o0, o1,
               isem0, isem1, wsem0, wsem1):
    wid = lax.axis_index("s") * NC + lax.axis_index("c")
    iota = lax.iota(jnp.int32, 16)
    gv_a = lax.shift_right_logical(iota, 3)
    gv_b = gv_a + 2
    er_v = lax.bitwise_and(iota, 7)

    def table(w2, out):
        nu = (NCH - wid + NW - 1) // NW

        def stage(v0, vc, sbuf, sem):
            for g in range(4):
                pltpu.async_copy(w2.at[pl.ds(g * 8, 8), pl.ds(v0, vc)],
                                 sbuf.at[g, :, pl.ds(0, vc)], sem)

        def drain_stage(vc, sem):
            for g in range(4):
                pltpu.make_async_copy(
                    w2.at[pl.ds(0, 8), pl.ds(0, vc)],
                    s0.at[0, :, pl.ds(0, vc)], sem).wait()

        def transpose(vc, sbuf, ob):
            def tbody(dv, carry):
                vv = iota * 0 + dv
                va = plsc.load_gather(sbuf, [gv_a, er_v, vv])
                vb = plsc.load_gather(sbuf, [gv_b, er_v, vv])
                r = dv >> 2
                c0 = (dv & 3) * EMB
                ob[r, pl.ds(c0, 16)] = va
                ob[r, pl.ds(c0 + 16, 16)] = vb
                return carry

            lax.fori_loop(0, vc, tbody, 0, unroll=False)

        def drain_write(rows, sem):
            pltpu.make_async_copy(
                o0.at[pl.ds(0, rows), :], out.at[pl.ds(0, rows), :],
                sem).wait()

        def chunk_i(i):
            return pl.multiple_of((wid + i * NW) * VC, 128)

        stage(chunk_i(0), VC, s0, isem0)

        def body(k, carry):
            for half, sbuf, ob, isem, wsem, sb_n, isem_n in (
                    (0, s0, o0, isem0, wsem0, s1, isem1),
                    (1, s1, o1, isem1, wsem1, s0, isem0)):
                i = 2 * k + half

                @pl.when(i < nu)
                def _():
                    drain_stage(VC, isem)

                    @pl.when(i + 1 < nu)
                    def _():
                        stage(chunk_i(i + 1), VC, sb_n, isem_n)

                    @pl.when(i >= 2)
                    def _():
                        drain_write(OR, wsem)

                    transpose(VC, sbuf, ob)
                    pltpu.async_copy(
                        ob,
                        out.at[pl.ds(pl.multiple_of(
                            (wid + i * NW) * OR, 8), OR), :],
                        wsem)
            return carry

        lax.fori_loop(0, (nu + 1) // 2, body, 0, unroll=False)
        drain_write(OR, wsem0)
        drain_write(OR, wsem1)

    table(wct, outc)
    table(wbt, outb)

    for tw, tail, out in ((0, tailc, outc), (1, tailb, outb)):
        @pl.when(wid == tw)
        def _():
            pltpu.async_copy(
                tail, out.at[pl.ds(NCH * VC * EMB // 128, TROWS), :], wsem0)
            pltpu.make_async_copy(
                tail, out.at[pl.ds(0, TROWS), :], wsem0).wait()


def _lk_kernel(idx_hbm, wc, wb, out_hbm,
               idx_a, idx_b, ric0, ric1, rib0, rib1, ac0, ac1, ab0, ab1,
               dc0, dc1, db0, db1, ob0, ob1, isem, gsem0, gsem1, wsem0,
               wsem1):
    wid = lax.axis_index("s") * NC + lax.axis_index("c")
    iota = lax.iota(jnp.int32, 16)

    def do_unit(u, carry):
        p = u // L
        l = u % L
        ia = pltpu.async_copy(
            idx_hbm.at[2 * p, pl.ds(pl.multiple_of(l * 32, 8), 32), :],
            idx_a, isem)
        ib = pltpu.async_copy(
            idx_hbm.at[2 * p + 1, pl.ds(pl.multiple_of(l * 32, 8), 32), :],
            idx_b, isem)
        ia.wait()
        ib.wait()

        def prep(q, ric, rib, ac, ab):
            def pbody(j, carry):
                vc = idx_a[q, pl.ds(j * 16, 16)]
                vb = idx_b[q, pl.ds(j * 16, 16)]
                ric[pl.ds(j * 16, 16)] = lax.shift_right_logical(vc, 2)
                rib[pl.ds(j * 16, 16)] = lax.shift_right_logical(vb, 2)
                ac[pl.ds(j * 16, 16)] = lax.shift_left(
                    lax.bitwise_and(vc, 3), 5)
                ab[pl.ds(j * 16, 16)] = lax.shift_left(
                    lax.bitwise_and(vb, 3), 5)
                return carry

            lax.fori_loop(0, BC // 16, pbody, 0, unroll=False)

        def fire(ric, rib, dc, db, sem):
            pltpu.async_copy(wc.at[ric], dc, sem)
            pltpu.async_copy(wb.at[rib], db, sem)

        def drain_gathers(sem):
            pltpu.make_async_copy(wc.at[pl.ds(0, BC), :], dc0, sem).wait()
            pltpu.make_async_copy(wc.at[pl.ds(0, BC), :], dc0, sem).wait()

        def extract(ac, ab, dc, db, ob):
            def ebody(j, carry):
                rv = iota + j * 16
                va = ac[pl.ds(j * 16, 16)]
                vb = ab[pl.ds(j * 16, 16)]
                for e in range(EMB):
                    ob[e, pl.ds(j * 16, 16)] = plsc.load_gather(
                        dc, [rv, va + e])
                for e in range(EMB):
                    ob[EMB + e, pl.ds(j * 16, 16)] = plsc.load_gather(
                        db, [rv, vb + e])
                return carry

            lax.fori_loop(0, BC // 16, ebody, 0, unroll=False)

        def write(q, ob, sem):
            pltpu.async_copy(
                ob,
                out_hbm.at[p, l, :, pl.ds(pl.multiple_of(q * BC, 128), BC)],
                sem)

        def drain_write(sem):
            pltpu.make_async_copy(
                ob0, out_hbm.at[0, 0, :, pl.ds(0, BC)], sem).wait()

        prep(0, ric0, rib0, ac0, ab0)
        fire(ric0, rib0, dc0, db0, gsem0)

        def chunk_pair(m, carry):
            q0 = 2 * m
            prep(q0 + 1, ric1, rib1, ac1, ab1)
            fire(ric1, rib1, dc1, db1, gsem1)
            drain_gathers(gsem0)

            @pl.when(m >= 1)
            def _():
                drain_write(wsem0)

            extract(ac0, ab0, dc0, db0, ob0)
            write(q0, ob0, wsem0)

            @pl.when(q0 + 2 < NBC)
            def _():
                prep(q0 + 2, ric0, rib0, ac0, ab0)
                fire(ric0, rib0, dc0, db0, gsem0)

            drain_gathers(gsem1)

            @pl.when(m >= 1)
            def _():
                drain_write(wsem1)

            extract(ac1, ab1, dc1, db1, ob1)
            write(q0 + 1, ob1, wsem1)
            return carry

        lax.fori_loop(0, NBC // 2, chunk_pair, 0, unroll=False)
        drain_write(wsem0)
        drain_write(wsem1)
        return carry

    nu = (NUNIT - wid + NW - 1) // NW

    def body(k, carry):
        return do_unit(wid + k * NW, carry)

    lax.fori_loop(0, nu, body, 0, unroll=False)


@jax.jit
def _run(idx_all, wct, wbt, tailc, tailb):
    mesh = plsc.VectorSubcoreMesh(core_axis_name="c", subcore_axis_name="s")
    tr = pl.kernel(
        _tr_kernel,
        out_type=(jax.ShapeDtypeStruct((NROW, 128), jnp.float32),
                  jax.ShapeDtypeStruct((NROW, 128), jnp.float32)),
        mesh=mesh,
        scratch_types=[
            pltpu.VMEM((4, 8, VC), jnp.float32),
            pltpu.VMEM((4, 8, VC), jnp.float32),
            pltpu.VMEM((OR, 128), jnp.float32),
            pltpu.VMEM((OR, 128), jnp.float32),
            pltpu.SemaphoreType.DMA,
            pltpu.SemaphoreType.DMA,
            pltpu.SemaphoreType.DMA,
            pltpu.SemaphoreType.DMA,
        ],
        compiler_params=_params,
    )
    wc, wb = tr(wct, wbt, tailc, tailb)
    lk = pl.kernel(
        _lk_kernel,
        out_type=jax.ShapeDtypeStruct((NPAIR, L, 2 * EMB, B), jnp.float32),
        mesh=mesh,
        scratch_types=[
            pltpu.VMEM((32, BC), jnp.int32),
            pltpu.VMEM((32, BC), jnp.int32),
            pltpu.VMEM((BC,), jnp.int32),
            pltpu.VMEM((BC,), jnp.int32),
            pltpu.VMEM((BC,), jnp.int32),
            pltpu.VMEM((BC,), jnp.int32),
            pltpu.VMEM((BC,), jnp.int32),
            pltpu.VMEM((BC,), jnp.int32),
            pltpu.VMEM((BC,), jnp.int32),
            pltpu.VMEM((BC,), jnp.int32),
            pltpu.VMEM((BC, 128), jnp.float32),
            pltpu.VMEM((BC, 128), jnp.float32),
            pltpu.VMEM((BC, 128), jnp.float32),
            pltpu.VMEM((BC, 128), jnp.float32),
            pltpu.VMEM((2 * EMB, BC), jnp.float32),
            pltpu.VMEM((2 * EMB, BC), jnp.float32),
            pltpu.SemaphoreType.DMA,
            pltpu.SemaphoreType.DMA,
            pltpu.SemaphoreType.DMA,
            pltpu.SemaphoreType.DMA,
            pltpu.SemaphoreType.DMA,
        ],
        compiler_params=_params,
    )
    return lk(idx_all, wc, wb)


def kernel(idx0, idx1, idx2, idx3, idx4, idx5, idx6, idx7, idx8, idx9,
           W_cate, W_brand):
    idx_all = jnp.stack(
        [i.T.reshape(L * 32, 128) for i in
         (idx0, idx1, idx2, idx3, idx4, idx5, idx6, idx7, idx8, idx9)])
    tailc = W_cate[NCH * VC:, :].reshape(TROWS, 128)
    tailb = W_brand[NCH * VC:, :].reshape(TROWS, 128)
    out_k = _run(idx_all, W_cate.T, W_brand.T, tailc, tailb)
    return jnp.transpose(out_k, (0, 3, 1, 2))
